# speedup vs baseline: 1.5263x; 1.5263x over previous
import functools

import jax
import jax.numpy as jnp
from jax import lax
from jax.experimental import pallas as pl
from jax.experimental.pallas import tpu as pltpu
from jax.experimental.pallas import tpu_sc as plsc

_BATCH = 16384
_HID = 1024
_NC = 2
_NS = 16
_NW = _NC * _NS
_B_PER_W = _BATCH // _NW
_CHUNK = 64
_NCHUNK = _B_PER_W // _CHUNK


def _make_gather():
    mesh = plsc.VectorSubcoreMesh(core_axis_name="c", subcore_axis_name="s")

    @functools.partial(
        pl.kernel,
        mesh=mesh,
        out_type=jax.ShapeDtypeStruct((_BATCH, _HID), jnp.float32),
        scratch_types=[
            pltpu.VMEM((_B_PER_W,), jnp.int32),
            pltpu.VMEM((_CHUNK, _HID), jnp.float32),
            pltpu.SemaphoreType.DMA,
        ],
    )
    def gather_kernel(table_hbm, idx_hbm, out_hbm, idx_v, rows_v, sem):
        wid = lax.axis_index("s") * _NC + lax.axis_index("c")
        base = wid * _B_PER_W
        pltpu.sync_copy(idx_hbm.at[pl.ds(base, _B_PER_W)], idx_v)
        for c in range(_NCHUNK):
            pltpu.async_copy(
                table_hbm.at[idx_v.at[pl.ds(c * _CHUNK, _CHUNK)]], rows_v, sem
            ).wait()
            pltpu.sync_copy(rows_v, out_hbm.at[pl.ds(base + c * _CHUNK, _CHUNK)])

    return gather_kernel


_gather = _make_gather()


@jax.jit
def kernel(x, condition, timestep, time_embed):
    return _gather(time_embed, timestep)


# speedup vs baseline: 1.6060x; 1.0522x over previous
import functools

import jax
import jax.numpy as jnp
from jax import lax
from jax.experimental import pallas as pl
from jax.experimental.pallas import tpu as pltpu
from jax.experimental.pallas import tpu_sc as plsc

_BATCH = 16384
_HID = 1024
_NC = 2
_NS = 16
_NW = _NC * _NS
_B_PER_W = _BATCH // _NW
_CHUNK = 32
_NCHUNK = _B_PER_W // _CHUNK


def _make_gather():
    mesh = plsc.VectorSubcoreMesh(core_axis_name="c", subcore_axis_name="s")

    @functools.partial(
        pl.kernel,
        mesh=mesh,
        out_type=jax.ShapeDtypeStruct((_BATCH, _HID), jnp.float32),
        scratch_types=[
            pltpu.VMEM((_B_PER_W,), jnp.int32),
            pltpu.VMEM((_CHUNK, _HID), jnp.float32),
            pltpu.VMEM((_CHUNK, _HID), jnp.float32),
            pltpu.SemaphoreType.DMA,
            pltpu.SemaphoreType.DMA,
            pltpu.SemaphoreType.DMA,
            pltpu.SemaphoreType.DMA,
        ],
    )
    def gather_kernel(table_hbm, idx_hbm, out_hbm, idx_v, rows0, rows1,
                      gsem0, gsem1, wsem0, wsem1):
        wid = lax.axis_index("s") * _NC + lax.axis_index("c")
        base = wid * _B_PER_W
        rows = (rows0, rows1)
        gsem = (gsem0, gsem1)
        wsem = (wsem0, wsem1)
        pltpu.sync_copy(idx_hbm.at[pl.ds(base, _B_PER_W)], idx_v)

        def gather(c):
            b = c & 1
            return pltpu.async_copy(
                table_hbm.at[idx_v.at[pl.ds(c * _CHUNK, _CHUNK)]],
                rows[b], gsem[b],
            )

        def writeback(c):
            b = c & 1
            return pltpu.async_copy(
                rows[b], out_hbm.at[pl.ds(base + c * _CHUNK, _CHUNK)], wsem[b]
            )

        g = [None, None]
        w = [None, None]
        g[0] = gather(0)
        for c in range(_NCHUNK):
            b = c & 1
            nb = 1 - b
            if c + 1 < _NCHUNK:
                if w[nb] is not None:
                    w[nb].wait()
                    w[nb] = None
                g[nb] = gather(c + 1)
            g[b].wait()
            w[b] = writeback(c)
        for h in w:
            if h is not None:
                h.wait()

    return gather_kernel


_gather = _make_gather()


@jax.jit
def kernel(x, condition, timestep, time_embed):
    return _gather(time_embed, timestep)


# speedup vs baseline: 1.6175x; 1.0071x over previous
import functools

import jax
import jax.numpy as jnp
from jax import lax
from jax.experimental import pallas as pl
from jax.experimental.pallas import tpu as pltpu
from jax.experimental.pallas import tpu_sc as plsc

_BATCH = 16384
_HID = 1024
_NC = 2
_NS = 16
_NW = _NC * _NS
_B_PER_W = _BATCH // _NW
_CHUNK = 32
_NCHUNK = _B_PER_W // _CHUNK
_NBUF = 3


def _make_gather():
    mesh = plsc.VectorSubcoreMesh(core_axis_name="c", subcore_axis_name="s")

    scratch = [pltpu.VMEM((_B_PER_W,), jnp.int32)]
    scratch += [pltpu.VMEM((_CHUNK, _HID), jnp.float32) for _ in range(_NBUF)]
    scratch += [pltpu.SemaphoreType.DMA for _ in range(2 * _NBUF)]

    @functools.partial(
        pl.kernel,
        mesh=mesh,
        out_type=jax.ShapeDtypeStruct((_BATCH, _HID), jnp.float32),
        scratch_types=scratch,
    )
    def gather_kernel(table_hbm, idx_hbm, out_hbm, idx_v, *bufs):
        rows = bufs[:_NBUF]
        gsem = bufs[_NBUF:2 * _NBUF]
        wsem = bufs[2 * _NBUF:]
        wid = lax.axis_index("s") * _NC + lax.axis_index("c")
        base = wid * _B_PER_W
        pltpu.sync_copy(idx_hbm.at[pl.ds(base, _B_PER_W)], idx_v)

        def gather(c):
            b = c % _NBUF
            return pltpu.async_copy(
                table_hbm.at[idx_v.at[pl.ds(c * _CHUNK, _CHUNK)]],
                rows[b], gsem[b],
            )

        def writeback(c):
            b = c % _NBUF
            return pltpu.async_copy(
                rows[b], out_hbm.at[pl.ds(base + c * _CHUNK, _CHUNK)], wsem[b]
            )

        g = {}
        w = {}
        for c in range(_NBUF - 1):
            g[c] = gather(c)
        for c in range(_NCHUNK):
            if c + _NBUF - 1 < _NCHUNK:
                if c - 1 in w:
                    w.pop(c - 1).wait()
                g[c + _NBUF - 1] = gather(c + _NBUF - 1)
            g.pop(c).wait()
            w[c] = writeback(c)
        for c in sorted(w):
            w.pop(c).wait()

    return gather_kernel


_gather = _make_gather()


@jax.jit
def kernel(x, condition, timestep, time_embed):
    return _gather(time_embed, timestep)
